# baseline (speedup 1.0000x reference)
import jax
import jax.numpy as jnp
from jax import lax
from jax.experimental import pallas as pl
from jax.experimental.pallas import tpu as pltpu
from jax.experimental.pallas import tpu_sc as plsc

N = 10000
D = 128
E = 320000
C = 64

NC = 2
NS = 16
L = 16
DH = D // NC

CHUNK = 128
ROWS = 160
EPT = ROWS * CHUNK
ETOT = NS * EPT

RPT = 640
NPAD = NS * RPT
NCHUNK = RPT // CHUNK
VPR = DH // L


def _rsqrt_newton(x):
    i = plsc.bitcast(x, jnp.int32)
    i = jnp.int32(0x5F3759DF) - (i >> 1)
    y = plsc.bitcast(i, jnp.float32)
    for _ in range(3):
        y = y * (jnp.float32(1.5) - jnp.float32(0.5) * x * y * y)
    return y


def _sgc_body(xh, srcp, dstp, out, g_hbm,
              acc_sp, deg_sp,
              src_idx, dst_idx, degloc, disloc,
              rowbuf0, rowbuf1, zbuf64, zbuf16, onesbuf,
              gsem0, gsem1, ssem0, ssem1, dsem):
    c = lax.axis_index("c")
    s = lax.axis_index("s")
    nbase = s * RPT
    gbase = c * NPAD + nbase

    zero16 = jnp.zeros((L,), jnp.float32)
    e0 = jnp.where(lax.iota(jnp.int32, L) == 0, jnp.float32(1.0),
                   jnp.float32(0.0))

    def _init_row(i, _):
        zbuf16[i, :] = zero16
        onesbuf[i, :] = e0
        for v in range(VPR):
            zbuf64[i, pl.ds(v * L, L)] = zero16
        return 0

    lax.fori_loop(0, CHUNK, _init_row, 0)

    def _zero_chunk(k, _):
        pltpu.sync_copy(zbuf16, deg_sp.at[pl.ds(nbase + k * CHUNK, CHUNK)])
        pltpu.sync_copy(zbuf64, acc_sp.at[pl.ds(nbase + k * CHUNK, CHUNK)])
        return 0

    lax.fori_loop(0, NCHUNK, _zero_chunk, 0)

    pltpu.sync_copy(srcp.at[c, pl.ds(s * ROWS, ROWS)], src_idx)
    pltpu.sync_copy(dstp.at[pl.ds(s * ROWS, ROWS)], dst_idx)

    plsc.subcore_barrier()

    def _deg_fire(j, _):
        pltpu.async_copy(onesbuf, deg_sp.at[dst_idx.at[j]], dsem, add=True)
        return 0

    def _deg_drain(j, _):
        pltpu.make_async_copy(onesbuf, deg_sp.at[dst_idx.at[j]], dsem).wait()
        return 0

    with jax.named_scope("deg"):
        lax.fori_loop(0, ROWS, _deg_fire, 0)
        lax.fori_loop(0, ROWS, _deg_drain, 0)

    plsc.subcore_barrier()

    def _dis_chunk(k, _):
        pltpu.sync_copy(deg_sp.at[pl.ds(nbase + k * CHUNK, CHUNK)], degloc)

        def _grp(g, _):
            ridx = g * L + lax.iota(jnp.int32, L)
            cidx = jnp.zeros((L,), jnp.int32)
            cnt = plsc.load_gather(degloc, [ridx, cidx])
            disloc[pl.ds(k * CHUNK + g * L, L)] = _rsqrt_newton(
                cnt + jnp.float32(1.0))
            return 0

        lax.fori_loop(0, CHUNK // L, _grp, 0)
        return 0

    lax.fori_loop(0, NCHUNK, _dis_chunk, 0)

    def _dis_splat(r):
        return plsc.load_gather(disloc, [jnp.full((L,), r, jnp.int32)])

    def _g0_chunk(k, _):
        base = k * CHUNK
        pltpu.sync_copy(xh.at[c, pl.ds(nbase + base, CHUNK)], rowbuf0)

        def _row(i, _):
            d = _dis_splat(base + i)
            for v in range(VPR):
                sl = pl.ds(v * L, L)
                rowbuf0[i, sl] = d * rowbuf0[i, sl]
            return 0

        lax.fori_loop(0, CHUNK, _row, 0)
        pltpu.sync_copy(rowbuf0, g_hbm.at[pl.ds(gbase + base, CHUNK)])
        return 0

    with jax.named_scope("dis_g0"):
        lax.fori_loop(0, NCHUNK, _g0_chunk, 0)

    plsc.subcore_barrier()

    def _edge_round():
        pltpu.async_copy(g_hbm.at[src_idx.at[0]], rowbuf0, gsem0)
        pltpu.async_copy(g_hbm.at[src_idx.at[1]], rowbuf1, gsem1)

        def _pair(k, _):
            j0 = 2 * k
            j1 = j0 + 1
            pltpu.make_async_copy(g_hbm.at[src_idx.at[j0]], rowbuf0,
                                  gsem0).wait()
            pltpu.async_copy(rowbuf0, acc_sp.at[dst_idx.at[j0]], ssem0,
                             add=True)
            pltpu.make_async_copy(g_hbm.at[src_idx.at[j1]], rowbuf1,
                                  gsem1).wait()
            pltpu.async_copy(rowbuf1, acc_sp.at[dst_idx.at[j1]], ssem1,
                             add=True)

            @pl.when(k < ROWS // 2 - 1)
            def _refill():
                pltpu.make_async_copy(rowbuf0, acc_sp.at[dst_idx.at[j0]],
                                      ssem0).wait()
                pltpu.async_copy(g_hbm.at[src_idx.at[j0 + 2]], rowbuf0, gsem0)
                pltpu.make_async_copy(rowbuf1, acc_sp.at[dst_idx.at[j1]],
                                      ssem1).wait()
                pltpu.async_copy(g_hbm.at[src_idx.at[j1 + 2]], rowbuf1, gsem1)

            return 0

        lax.fori_loop(0, ROWS // 2, _pair, 0)
        pltpu.make_async_copy(rowbuf0, acc_sp.at[dst_idx.at[0]], ssem0).wait()
        pltpu.make_async_copy(rowbuf1, acc_sp.at[dst_idx.at[1]], ssem1).wait()

    with jax.named_scope("hop1"):
        _edge_round()

    plsc.subcore_barrier()

    def _g1_chunk(k, _):
        base = k * CHUNK
        pltpu.sync_copy(acc_sp.at[pl.ds(nbase + base, CHUNK)], rowbuf1)
        pltpu.sync_copy(g_hbm.at[pl.ds(gbase + base, CHUNK)], rowbuf0)

        def _row(i, _):
            d = _dis_splat(base + i)
            d2 = d * d
            for v in range(VPR):
                sl = pl.ds(v * L, L)
                rowbuf0[i, sl] = d2 * (rowbuf1[i, sl] + rowbuf0[i, sl])
            return 0

        lax.fori_loop(0, CHUNK, _row, 0)
        pltpu.sync_copy(rowbuf0, g_hbm.at[pl.ds(gbase + base, CHUNK)])
        pltpu.sync_copy(zbuf64, acc_sp.at[pl.ds(nbase + base, CHUNK)])
        return 0

    with jax.named_scope("g1"):
        lax.fori_loop(0, NCHUNK, _g1_chunk, 0)

    plsc.subcore_barrier()

    with jax.named_scope("hop2"):
        _edge_round()

    plsc.subcore_barrier()

    def _out_chunk(k, _):
        base = k * CHUNK
        pltpu.sync_copy(acc_sp.at[pl.ds(nbase + base, CHUNK)], rowbuf1)
        pltpu.sync_copy(g_hbm.at[pl.ds(gbase + base, CHUNK)], rowbuf0)

        def _row(i, _):
            d = _dis_splat(base + i)
            for v in range(VPR):
                sl = pl.ds(v * L, L)
                rowbuf1[i, sl] = d * (rowbuf1[i, sl] + rowbuf0[i, sl])
            return 0

        lax.fori_loop(0, CHUNK, _row, 0)
        pltpu.sync_copy(rowbuf1, out.at[c, pl.ds(nbase + base, CHUNK)])
        return 0

    with jax.named_scope("outp"):
        lax.fori_loop(0, NCHUNK, _out_chunk, 0)


def _propagate_sc(xh, srcp, dstp):
    mesh = plsc.VectorSubcoreMesh(core_axis_name="c", subcore_axis_name="s")
    out, _ = pl.kernel(
        _sgc_body,
        out_type=(
            jax.ShapeDtypeStruct((NC, NPAD, DH), jnp.float32),
            jax.ShapeDtypeStruct((NC * NPAD, DH), jnp.float32),
        ),
        mesh=mesh,
        compiler_params=pltpu.CompilerParams(needs_layout_passes=False,
                                             use_tc_tiling_on_sc=False),
        scratch_types=[
            pltpu.VMEM_SHARED((NPAD, DH), jnp.float32),
            pltpu.VMEM_SHARED((NPAD, L), jnp.float32),
            pltpu.VMEM((ROWS, CHUNK), jnp.int32),
            pltpu.VMEM((ROWS, CHUNK), jnp.int32),
            pltpu.VMEM((CHUNK, L), jnp.float32),
            pltpu.VMEM((RPT,), jnp.float32),
            pltpu.VMEM((CHUNK, DH), jnp.float32),
            pltpu.VMEM((CHUNK, DH), jnp.float32),
            pltpu.VMEM((CHUNK, DH), jnp.float32),
            pltpu.VMEM((CHUNK, L), jnp.float32),
            pltpu.VMEM((CHUNK, L), jnp.float32),
            pltpu.SemaphoreType.DMA,
            pltpu.SemaphoreType.DMA,
            pltpu.SemaphoreType.DMA,
            pltpu.SemaphoreType.DMA,
            pltpu.SemaphoreType.DMA,
        ],
    )(xh, srcp, dstp)
    return out


def _linsoftmax_body(h_ref, wt_ref, b_ref, o_ref):
    logits = jnp.dot(h_ref[...], wt_ref[...],
                     preferred_element_type=jnp.float32) + b_ref[...]
    m = jnp.max(logits, axis=1, keepdims=True)
    ex = jnp.exp(logits - m)
    o_ref[...] = ex / jnp.sum(ex, axis=1, keepdims=True)


def _linsoftmax_tc(h, wt, b2):
    blk = 1000
    grid = N // blk
    return pl.pallas_call(
        _linsoftmax_body,
        grid=(grid,),
        in_specs=[
            pl.BlockSpec((blk, D), lambda i: (i, 0)),
            pl.BlockSpec((D, C), lambda i: (0, 0)),
            pl.BlockSpec((1, C), lambda i: (0, 0)),
        ],
        out_specs=pl.BlockSpec((blk, C), lambda i: (i, 0)),
        out_shape=jax.ShapeDtypeStruct((N, C), jnp.float32),
    )(h, wt, b2)


def kernel(x, edge_index, W, b):
    src = edge_index[0]
    dst = edge_index[1]
    pad = jnp.full((ETOT - E,), N, dtype=jnp.int32)
    src1 = jnp.concatenate([src, pad])
    srcp = jnp.stack([src1, src1 + NPAD]).reshape(NC, NS * ROWS, CHUNK)
    dstp = jnp.concatenate([dst, pad]).reshape(NS * ROWS, CHUNK)
    xp = jnp.pad(x, ((0, NPAD - N), (0, 0)))
    xh = xp.reshape(NPAD, NC, DH).transpose(1, 0, 2)

    halves = _propagate_sc(xh, srcp, dstp)
    h2 = halves[:, :N].transpose(1, 0, 2).reshape(N, D)

    return _linsoftmax_tc(h2, W.T, b.reshape(1, C))


# speedup vs baseline: 1.0038x; 1.0038x over previous
import jax
import jax.numpy as jnp
from jax import lax
from jax.experimental import pallas as pl
from jax.experimental.pallas import tpu as pltpu
from jax.experimental.pallas import tpu_sc as plsc

N = 10000
D = 128
E = 320000
C = 64

NC = 2
NS = 16
L = 16
DH = D // NC

CHUNK = 128
ROWS = 160
EPT = ROWS * CHUNK
ETOT = NS * EPT

RPT = 640
NPAD = NS * RPT
NCHUNK = RPT // CHUNK
VPR = DH // L


def _rsqrt_newton(x):
    i = plsc.bitcast(x, jnp.int32)
    i = jnp.int32(0x5F3759DF) - (i >> 1)
    y = plsc.bitcast(i, jnp.float32)
    for _ in range(3):
        y = y * (jnp.float32(1.5) - jnp.float32(0.5) * x * y * y)
    return y


def _sgc_body(xh, srcp, dstp, out, g_hbm,
              acc_sp, deg_sp,
              src_idx, dst_idx, degloc, disloc,
              rowbuf0, rowbuf1, zbuf64, zbuf16, onesbuf,
              gsem0, gsem1, ssem0, ssem1, dsem):
    c = lax.axis_index("c")
    s = lax.axis_index("s")
    nbase = s * RPT
    gbase = c * NPAD + nbase
    scope = jax.named_scope

    zero16 = jnp.zeros((L,), jnp.float32)
    e0 = jnp.where(lax.iota(jnp.int32, L) == 0, jnp.float32(1.0),
                   jnp.float32(0.0))

    _sc_init = scope("init")
    _sc_init.__enter__()

    def _init_row(i, _):
        zbuf16[i, :] = zero16
        onesbuf[i, :] = e0
        for v in range(VPR):
            zbuf64[i, pl.ds(v * L, L)] = zero16
        return 0

    lax.fori_loop(0, CHUNK, _init_row, 0)

    def _zero_chunk(k, _):
        pltpu.sync_copy(zbuf16, deg_sp.at[pl.ds(nbase + k * CHUNK, CHUNK)])
        pltpu.sync_copy(zbuf64, acc_sp.at[pl.ds(nbase + k * CHUNK, CHUNK)])
        return 0

    lax.fori_loop(0, NCHUNK, _zero_chunk, 0)

    pltpu.sync_copy(srcp.at[c, pl.ds(s * ROWS, ROWS)], src_idx)
    pltpu.sync_copy(dstp.at[pl.ds(s * ROWS, ROWS)], dst_idx)
    _sc_init.__exit__(None, None, None)

    with scope("bar0"):
        plsc.subcore_barrier()

    def _deg_fire(j, _):
        pltpu.async_copy(onesbuf, deg_sp.at[dst_idx.at[j]], dsem, add=True)
        return 0

    def _deg_drain(j, _):
        pltpu.make_async_copy(onesbuf, deg_sp.at[dst_idx.at[j]], dsem).wait()
        return 0

    with jax.named_scope("deg"):
        lax.fori_loop(0, ROWS, _deg_fire, 0)
        lax.fori_loop(0, ROWS, _deg_drain, 0)

    with scope("bar1"):
        plsc.subcore_barrier()

    def _dis_chunk(k, _):
        pltpu.sync_copy(deg_sp.at[pl.ds(nbase + k * CHUNK, CHUNK)], degloc)

        def _grp(g, _):
            ridx = g * L + lax.iota(jnp.int32, L)
            cidx = jnp.zeros((L,), jnp.int32)
            cnt = plsc.load_gather(degloc, [ridx, cidx])
            disloc[pl.ds(k * CHUNK + g * L, L)] = _rsqrt_newton(
                cnt + jnp.float32(1.0))
            return 0

        lax.fori_loop(0, CHUNK // L, _grp, 0)
        return 0

    lax.fori_loop(0, NCHUNK, _dis_chunk, 0)

    def _dis_splat(r):
        return plsc.load_gather(disloc, [jnp.full((L,), r, jnp.int32)])

    def _g0_chunk(k, _):
        base = k * CHUNK
        pltpu.sync_copy(xh.at[c, pl.ds(nbase + base, CHUNK)], rowbuf0)

        def _row(i, _):
            d = _dis_splat(base + i)
            for v in range(VPR):
                sl = pl.ds(v * L, L)
                rowbuf0[i, sl] = d * rowbuf0[i, sl]
            return 0

        lax.fori_loop(0, CHUNK, _row, 0)
        pltpu.sync_copy(rowbuf0, g_hbm.at[pl.ds(gbase + base, CHUNK)])
        return 0

    with jax.named_scope("dis_g0"):
        lax.fori_loop(0, NCHUNK, _g0_chunk, 0)

    with scope("bar2"):
        plsc.subcore_barrier()

    def _edge_round():
        pltpu.async_copy(g_hbm.at[src_idx.at[0]], rowbuf0, gsem0)
        pltpu.async_copy(g_hbm.at[src_idx.at[1]], rowbuf1, gsem1)

        def _pair(k, _):
            j0 = 2 * k
            j1 = j0 + 1
            pltpu.make_async_copy(g_hbm.at[src_idx.at[j0]], rowbuf0,
                                  gsem0).wait()
            pltpu.async_copy(rowbuf0, acc_sp.at[dst_idx.at[j0]], ssem0,
                             add=True)
            pltpu.make_async_copy(g_hbm.at[src_idx.at[j1]], rowbuf1,
                                  gsem1).wait()
            pltpu.async_copy(rowbuf1, acc_sp.at[dst_idx.at[j1]], ssem1,
                             add=True)

            @pl.when(k < ROWS // 2 - 1)
            def _refill():
                pltpu.make_async_copy(rowbuf0, acc_sp.at[dst_idx.at[j0]],
                                      ssem0).wait()
                pltpu.async_copy(g_hbm.at[src_idx.at[j0 + 2]], rowbuf0, gsem0)
                pltpu.make_async_copy(rowbuf1, acc_sp.at[dst_idx.at[j1]],
                                      ssem1).wait()
                pltpu.async_copy(g_hbm.at[src_idx.at[j1 + 2]], rowbuf1, gsem1)

            return 0

        lax.fori_loop(0, ROWS // 2, _pair, 0)
        pltpu.make_async_copy(rowbuf0, acc_sp.at[dst_idx.at[0]], ssem0).wait()
        pltpu.make_async_copy(rowbuf1, acc_sp.at[dst_idx.at[1]], ssem1).wait()

    with jax.named_scope("hop1"):
        _edge_round()

    with scope("bar3"):
        plsc.subcore_barrier()

    def _g1_chunk(k, _):
        base = k * CHUNK
        pltpu.sync_copy(acc_sp.at[pl.ds(nbase + base, CHUNK)], rowbuf1)
        pltpu.sync_copy(g_hbm.at[pl.ds(gbase + base, CHUNK)], rowbuf0)

        def _row(i, _):
            d = _dis_splat(base + i)
            d2 = d * d
            for v in range(VPR):
                sl = pl.ds(v * L, L)
                rowbuf0[i, sl] = d2 * (rowbuf1[i, sl] + rowbuf0[i, sl])
            return 0

        lax.fori_loop(0, CHUNK, _row, 0)
        pltpu.sync_copy(rowbuf0, g_hbm.at[pl.ds(gbase + base, CHUNK)])
        pltpu.sync_copy(zbuf64, acc_sp.at[pl.ds(nbase + base, CHUNK)])
        return 0

    with jax.named_scope("g1"):
        lax.fori_loop(0, NCHUNK, _g1_chunk, 0)

    with scope("bar4"):
        plsc.subcore_barrier()

    with jax.named_scope("hop2"):
        _edge_round()

    with scope("bar5"):
        plsc.subcore_barrier()

    def _out_chunk(k, _):
        base = k * CHUNK
        pltpu.sync_copy(acc_sp.at[pl.ds(nbase + base, CHUNK)], rowbuf1)
        pltpu.sync_copy(g_hbm.at[pl.ds(gbase + base, CHUNK)], rowbuf0)

        def _row(i, _):
            d = _dis_splat(base + i)
            for v in range(VPR):
                sl = pl.ds(v * L, L)
                rowbuf1[i, sl] = d * (rowbuf1[i, sl] + rowbuf0[i, sl])
            return 0

        lax.fori_loop(0, CHUNK, _row, 0)
        pltpu.sync_copy(rowbuf1, out.at[c, pl.ds(nbase + base, CHUNK)])
        return 0

    with jax.named_scope("outp"):
        lax.fori_loop(0, NCHUNK, _out_chunk, 0)


def _propagate_sc(xh, srcp, dstp):
    mesh = plsc.VectorSubcoreMesh(core_axis_name="c", subcore_axis_name="s")
    out, _ = pl.kernel(
        _sgc_body,
        out_type=(
            jax.ShapeDtypeStruct((NC, NPAD, DH), jnp.float32),
            jax.ShapeDtypeStruct((NC * NPAD, DH), jnp.float32),
        ),
        mesh=mesh,
        compiler_params=pltpu.CompilerParams(needs_layout_passes=False,
                                             use_tc_tiling_on_sc=False),
        scratch_types=[
            pltpu.VMEM_SHARED((NPAD, DH), jnp.float32),
            pltpu.VMEM_SHARED((NPAD, L), jnp.float32),
            pltpu.VMEM((ROWS, CHUNK), jnp.int32),
            pltpu.VMEM((ROWS, CHUNK), jnp.int32),
            pltpu.VMEM((CHUNK, L), jnp.float32),
            pltpu.VMEM((RPT,), jnp.float32),
            pltpu.VMEM((CHUNK, DH), jnp.float32),
            pltpu.VMEM((CHUNK, DH), jnp.float32),
            pltpu.VMEM((CHUNK, DH), jnp.float32),
            pltpu.VMEM((CHUNK, L), jnp.float32),
            pltpu.VMEM((CHUNK, L), jnp.float32),
            pltpu.SemaphoreType.DMA,
            pltpu.SemaphoreType.DMA,
            pltpu.SemaphoreType.DMA,
            pltpu.SemaphoreType.DMA,
            pltpu.SemaphoreType.DMA,
        ],
    )(xh, srcp, dstp)
    return out


def _linsoftmax_body(h_ref, wt_ref, b_ref, o_ref):
    logits = jnp.dot(h_ref[...], wt_ref[...],
                     preferred_element_type=jnp.float32) + b_ref[...]
    m = jnp.max(logits, axis=1, keepdims=True)
    ex = jnp.exp(logits - m)
    o_ref[...] = ex / jnp.sum(ex, axis=1, keepdims=True)


def _linsoftmax_tc(h, wt, b2):
    blk = 1000
    grid = N // blk
    return pl.pallas_call(
        _linsoftmax_body,
        grid=(grid,),
        in_specs=[
            pl.BlockSpec((blk, D), lambda i: (i, 0)),
            pl.BlockSpec((D, C), lambda i: (0, 0)),
            pl.BlockSpec((1, C), lambda i: (0, 0)),
        ],
        out_specs=pl.BlockSpec((blk, C), lambda i: (i, 0)),
        out_shape=jax.ShapeDtypeStruct((N, C), jnp.float32),
    )(h, wt, b2)


def kernel(x, edge_index, W, b):
    src = edge_index[0]
    dst = edge_index[1]
    pad = jnp.full((ETOT - E,), N, dtype=jnp.int32)
    src1 = jnp.concatenate([src, pad])
    srcp = jnp.stack([src1, src1 + NPAD]).reshape(NC, NS * ROWS, CHUNK)
    dstp = jnp.concatenate([dst, pad]).reshape(NS * ROWS, CHUNK)
    xp = jnp.pad(x, ((0, NPAD - N), (0, 0)))
    xh = xp.reshape(NPAD, NC, DH).transpose(1, 0, 2)

    halves = _propagate_sc(xh, srcp, dstp)
    h2 = halves[:, :N].transpose(1, 0, 2).reshape(N, D)

    return _linsoftmax_tc(h2, W.T, b.reshape(1, C))


# speedup vs baseline: 1.5121x; 1.5064x over previous
import jax
import jax.numpy as jnp
from jax import lax
from jax.experimental import pallas as pl
from jax.experimental.pallas import tpu as pltpu
from jax.experimental.pallas import tpu_sc as plsc

N = 10000
D = 128
E = 320000
C = 64

NC = 2
NS = 16
L = 16
DH = D // NC

CHUNK = 128
ROWS = 160
EPT = ROWS * CHUNK
ETOT = NS * EPT

RPT = 640
NPAD = NS * RPT
NCHUNK = RPT // CHUNK
VPR = DH // L


def _rsqrt_newton(x):
    i = plsc.bitcast(x, jnp.int32)
    i = jnp.int32(0x5F3759DF) - (i >> 1)
    y = plsc.bitcast(i, jnp.float32)
    for _ in range(3):
        y = y * (jnp.float32(1.5) - jnp.float32(0.5) * x * y * y)
    return y


def _sgc_body(xh, srcp, dstp, out, g_hbm,
              acc_sp, deg_sp,
              src_idx, dst_idx, degloc, disloc,
              rowbuf0, rowbuf1, zbuf64, zbuf16, onesbuf,
              gsem0, gsem1, ssem0, ssem1, dsem):
    c = lax.axis_index("c")
    s = lax.axis_index("s")
    nbase = s * RPT
    gbase = c * NPAD + nbase
    scope = jax.named_scope

    zero16 = jnp.zeros((L,), jnp.float32)
    e0 = jnp.where(lax.iota(jnp.int32, L) == 0, jnp.float32(1.0),
                   jnp.float32(0.0))

    _sc_init = scope("init")
    _sc_init.__enter__()

    def _init_row(i, _):
        zbuf16[i, :] = zero16
        onesbuf[i, :] = e0
        for v in range(VPR):
            zbuf64[i, pl.ds(v * L, L)] = zero16
        return 0

    lax.fori_loop(0, CHUNK, _init_row, 0)

    def _zero_chunk(k, _):
        pltpu.sync_copy(zbuf16, deg_sp.at[pl.ds(nbase + k * CHUNK, CHUNK)])
        pltpu.sync_copy(zbuf64, acc_sp.at[pl.ds(nbase + k * CHUNK, CHUNK)])
        return 0

    lax.fori_loop(0, NCHUNK, _zero_chunk, 0)

    pltpu.sync_copy(srcp.at[c, pl.ds(s * ROWS, ROWS)], src_idx)
    pltpu.sync_copy(dstp.at[pl.ds(s * ROWS, ROWS)], dst_idx)
    _sc_init.__exit__(None, None, None)

    with scope("bar0"):
        plsc.subcore_barrier()

    def _deg_fire(j, _):
        pltpu.async_copy(onesbuf, deg_sp.at[dst_idx.at[j]], dsem, add=True)
        return 0

    def _deg_drain(j, _):
        pltpu.make_async_copy(onesbuf, deg_sp.at[dst_idx.at[j]], dsem).wait()
        return 0

    with jax.named_scope("deg"):
        lax.fori_loop(0, ROWS, _deg_fire, 0)
        lax.fori_loop(0, ROWS, _deg_drain, 0)

    with scope("bar1"):
        plsc.subcore_barrier()

    def _dis_chunk(k, _):
        pltpu.sync_copy(deg_sp.at[pl.ds(nbase + k * CHUNK, CHUNK)], degloc)

        def _grp(g, _):
            ridx = g * L + lax.iota(jnp.int32, L)
            cidx = jnp.zeros((L,), jnp.int32)
            cnt = plsc.load_gather(degloc, [ridx, cidx])
            disloc[pl.ds(k * CHUNK + g * L, L)] = _rsqrt_newton(
                cnt + jnp.float32(1.0))
            return 0

        lax.fori_loop(0, CHUNK // L, _grp, 0)
        return 0

    lax.fori_loop(0, NCHUNK, _dis_chunk, 0)

    def _dis_splat(r):
        return plsc.load_gather(disloc, [jnp.full((L,), r, jnp.int32)])

    def _g0_chunk(k, _):
        base = k * CHUNK
        pltpu.sync_copy(xh.at[c, pl.ds(nbase + base, CHUNK)], rowbuf0)

        def _row(i, _):
            d = _dis_splat(base + i)
            for v in range(VPR):
                sl = pl.ds(v * L, L)
                rowbuf0[i, sl] = d * rowbuf0[i, sl]
            return 0

        lax.fori_loop(0, CHUNK, _row, 0)
        pltpu.sync_copy(rowbuf0, g_hbm.at[pl.ds(gbase + base, CHUNK)])
        return 0

    with jax.named_scope("dis_g0"):
        lax.fori_loop(0, NCHUNK, _g0_chunk, 0)

    with scope("bar2"):
        plsc.subcore_barrier()

    def _edge_round():
        pltpu.async_copy(g_hbm.at[src_idx.at[0]], rowbuf0, gsem0)
        pltpu.async_copy(g_hbm.at[src_idx.at[1]], rowbuf1, gsem1)

        def _pair(k, _):
            j0 = 2 * k
            j1 = j0 + 1
            pltpu.make_async_copy(g_hbm.at[src_idx.at[j0]], rowbuf0,
                                  gsem0).wait()
            pltpu.async_copy(rowbuf0, acc_sp.at[dst_idx.at[j0]], ssem0,
                             add=True)
            pltpu.make_async_copy(g_hbm.at[src_idx.at[j1]], rowbuf1,
                                  gsem1).wait()
            pltpu.async_copy(rowbuf1, acc_sp.at[dst_idx.at[j1]], ssem1,
                             add=True)

            @pl.when(k < ROWS // 2 - 1)
            def _refill():
                pltpu.make_async_copy(rowbuf0, acc_sp.at[dst_idx.at[j0]],
                                      ssem0).wait()
                pltpu.async_copy(g_hbm.at[src_idx.at[j0 + 2]], rowbuf0, gsem0)
                pltpu.make_async_copy(rowbuf1, acc_sp.at[dst_idx.at[j1]],
                                      ssem1).wait()
                pltpu.async_copy(g_hbm.at[src_idx.at[j1 + 2]], rowbuf1, gsem1)

            return 0

        lax.fori_loop(0, ROWS // 2, _pair, 0)
        pltpu.make_async_copy(rowbuf0, acc_sp.at[dst_idx.at[0]], ssem0).wait()
        pltpu.make_async_copy(rowbuf1, acc_sp.at[dst_idx.at[1]], ssem1).wait()

    with jax.named_scope("hop1"):
        _edge_round()

    with scope("bar3"):
        plsc.subcore_barrier()

    def _g1_chunk(k, _):
        base = k * CHUNK
        pltpu.sync_copy(acc_sp.at[pl.ds(nbase + base, CHUNK)], rowbuf1)
        pltpu.sync_copy(g_hbm.at[pl.ds(gbase + base, CHUNK)], rowbuf0)

        def _row(i, _):
            d = _dis_splat(base + i)
            d2 = d * d
            for v in range(VPR):
                sl = pl.ds(v * L, L)
                rowbuf0[i, sl] = d2 * (rowbuf1[i, sl] + rowbuf0[i, sl])
            return 0

        lax.fori_loop(0, CHUNK, _row, 0)
        pltpu.sync_copy(rowbuf0, g_hbm.at[pl.ds(gbase + base, CHUNK)])
        pltpu.sync_copy(zbuf64, acc_sp.at[pl.ds(nbase + base, CHUNK)])
        return 0

    with jax.named_scope("g1"):
        lax.fori_loop(0, NCHUNK, _g1_chunk, 0)

    with scope("bar4"):
        plsc.subcore_barrier()

    with jax.named_scope("hop2"):
        _edge_round()

    with scope("bar5"):
        plsc.subcore_barrier()

    def _out_chunk(k, _):
        base = k * CHUNK
        pltpu.sync_copy(acc_sp.at[pl.ds(nbase + base, CHUNK)], rowbuf1)
        pltpu.sync_copy(g_hbm.at[pl.ds(gbase + base, CHUNK)], rowbuf0)

        def _row(i, _):
            d = _dis_splat(base + i)
            for v in range(VPR):
                sl = pl.ds(v * L, L)
                rowbuf1[i, sl] = d * (rowbuf1[i, sl] + rowbuf0[i, sl])
            return 0

        lax.fori_loop(0, CHUNK, _row, 0)
        pltpu.sync_copy(rowbuf1, out.at[c, pl.ds(nbase + base, CHUNK)])
        return 0

    with jax.named_scope("outp"):
        lax.fori_loop(0, NCHUNK, _out_chunk, 0)


def _propagate_sc(xh, srcp, dstp):
    mesh = plsc.VectorSubcoreMesh(core_axis_name="c", subcore_axis_name="s")
    out, _ = pl.kernel(
        _sgc_body,
        out_type=(
            jax.ShapeDtypeStruct((NC, NPAD, DH), jnp.float32),
            jax.ShapeDtypeStruct((NC * NPAD, DH), jnp.float32),
        ),
        mesh=mesh,
        compiler_params=pltpu.CompilerParams(needs_layout_passes=False,
                                             use_tc_tiling_on_sc=False),
        scratch_types=[
            pltpu.VMEM_SHARED((NPAD, DH), jnp.float32),
            pltpu.VMEM_SHARED((NPAD, L), jnp.float32),
            pltpu.VMEM((ROWS, CHUNK), jnp.int32),
            pltpu.VMEM((ROWS, CHUNK), jnp.int32),
            pltpu.VMEM((CHUNK, L), jnp.float32),
            pltpu.VMEM((RPT,), jnp.float32),
            pltpu.VMEM((CHUNK, DH), jnp.float32),
            pltpu.VMEM((CHUNK, DH), jnp.float32),
            pltpu.VMEM((CHUNK, DH), jnp.float32),
            pltpu.VMEM((CHUNK, L), jnp.float32),
            pltpu.VMEM((CHUNK, L), jnp.float32),
            pltpu.SemaphoreType.DMA,
            pltpu.SemaphoreType.DMA,
            pltpu.SemaphoreType.DMA,
            pltpu.SemaphoreType.DMA,
            pltpu.SemaphoreType.DMA,
        ],
    )(xh, srcp, dstp)
    return out


def _linsoftmax_body(h_ref, wt_ref, b_ref, o_ref):
    logits = jnp.dot(h_ref[...], wt_ref[...],
                     preferred_element_type=jnp.float32) + b_ref[...]
    m = jnp.max(logits, axis=1, keepdims=True)
    ex = jnp.exp(logits - m)
    o_ref[...] = ex / jnp.sum(ex, axis=1, keepdims=True)


def _linsoftmax_tc(h, wt, b2):
    blk = 1000
    grid = N // blk
    return pl.pallas_call(
        _linsoftmax_body,
        grid=(grid,),
        in_specs=[
            pl.BlockSpec((blk, D), lambda i: (i, 0)),
            pl.BlockSpec((D, C), lambda i: (0, 0)),
            pl.BlockSpec((1, C), lambda i: (0, 0)),
        ],
        out_specs=pl.BlockSpec((blk, C), lambda i: (i, 0)),
        out_shape=jax.ShapeDtypeStruct((N, C), jnp.float32),
    )(h, wt, b2)


def kernel(x, edge_index, W, b):
    src = edge_index[0]
    dst = edge_index[1]
    pad = N + (jnp.arange(ETOT - E, dtype=jnp.int32) % (NPAD - N))
    src1 = jnp.concatenate([src, pad])
    srcp = jnp.stack([src1, src1 + NPAD]).reshape(NC, NS * ROWS, CHUNK)
    dstp = jnp.concatenate([dst, pad]).reshape(NS * ROWS, CHUNK)
    xp = jnp.pad(x, ((0, NPAD - N), (0, 0)))
    xh = xp.reshape(NPAD, NC, DH).transpose(1, 0, 2)

    halves = _propagate_sc(xh, srcp, dstp)
    h2 = halves[:, :N].transpose(1, 0, 2).reshape(N, D)

    return _linsoftmax_tc(h2, W.T, b.reshape(1, C))


# speedup vs baseline: 2.0238x; 1.3384x over previous
import jax
import jax.numpy as jnp
from jax import lax
from jax.experimental import pallas as pl
from jax.experimental.pallas import tpu as pltpu
from jax.experimental.pallas import tpu_sc as plsc

N = 10000
D = 128
E = 320000
C = 64

NC = 2
NS = 16
L = 16
DH = D // NC

CHUNK = 128
ROWS = 160
EPT = ROWS * CHUNK
ETOT = NS * EPT

RPT = 640
NPAD = NS * RPT
NCHUNK = RPT // CHUNK
VPR = DH // L


def _rsqrt_newton(x):
    i = plsc.bitcast(x, jnp.int32)
    i = jnp.int32(0x5F3759DF) - (i >> 1)
    y = plsc.bitcast(i, jnp.float32)
    for _ in range(3):
        y = y * (jnp.float32(1.5) - jnp.float32(0.5) * x * y * y)
    return y


def _sgc_body(xh, srcp, dstp, out, g_hbm,
              acc_sp, deg_sp,
              src_idx, dst_idx, degloc, disloc,
              rowbuf0, rowbuf1, rowbuf3, zbuf64, zbuf16, onesbuf,
              gsem0, gsem1, gsem2, gsem3, ssem0, ssem1, ssem2, ssem3, dsem):
    c = lax.axis_index("c")
    s = lax.axis_index("s")
    nbase = s * RPT
    gbase = c * NPAD + nbase
    scope = jax.named_scope

    zero16 = jnp.zeros((L,), jnp.float32)
    e0 = jnp.where(lax.iota(jnp.int32, L) == 0, jnp.float32(1.0),
                   jnp.float32(0.0))

    _sc_init = scope("init")
    _sc_init.__enter__()

    def _init_row(i, _):
        onesbuf[i, :] = e0
        for v in range(VPR):
            zbuf64[i, pl.ds(v * L, L)] = zero16
        return 0

    lax.fori_loop(0, CHUNK, _init_row, 0)

    def _init_z16(i, _):
        zbuf16[i, :] = zero16
        return 0

    lax.fori_loop(0, CHUNK // 2, _init_z16, 0)

    def _zero_chunk(k, _):
        pltpu.sync_copy(zbuf16,
                        deg_sp.at[pl.ds(nbase + k * (CHUNK // 2), CHUNK // 2)])
        return 0

    lax.fori_loop(0, 2 * NCHUNK, _zero_chunk, 0)

    def _zero_acc_chunk(k, _):
        pltpu.sync_copy(zbuf64, acc_sp.at[pl.ds(nbase + k * CHUNK, CHUNK)])
        return 0

    lax.fori_loop(0, NCHUNK, _zero_acc_chunk, 0)

    pltpu.sync_copy(srcp.at[c, pl.ds(s * ROWS, ROWS)], src_idx)
    pltpu.sync_copy(dstp.at[pl.ds(s * ROWS, ROWS)], dst_idx)
    _sc_init.__exit__(None, None, None)

    with scope("bar0"):
        plsc.subcore_barrier()

    def _deg_fire(j, _):
        pltpu.async_copy(onesbuf, deg_sp.at[dst_idx.at[j]], dsem, add=True)
        return 0

    def _deg_drain(j, _):
        pltpu.make_async_copy(onesbuf, deg_sp.at[dst_idx.at[j]], dsem).wait()
        return 0

    with jax.named_scope("deg"):
        lax.fori_loop(0, ROWS, _deg_fire, 0)
        lax.fori_loop(0, ROWS, _deg_drain, 0)

    with scope("bar1"):
        plsc.subcore_barrier()

    def _dis_chunk(k, _):
        pltpu.sync_copy(deg_sp.at[pl.ds(nbase + k * CHUNK, CHUNK)], degloc)

        def _grp(g, _):
            ridx = g * L + lax.iota(jnp.int32, L)
            cidx = jnp.zeros((L,), jnp.int32)
            cnt = plsc.load_gather(degloc, [ridx, cidx])
            disloc[pl.ds(k * CHUNK + g * L, L)] = _rsqrt_newton(
                cnt + jnp.float32(1.0))
            return 0

        lax.fori_loop(0, CHUNK // L, _grp, 0)
        return 0

    lax.fori_loop(0, NCHUNK, _dis_chunk, 0)

    def _dis_splat(r):
        return plsc.load_gather(disloc, [jnp.full((L,), r, jnp.int32)])

    def _g0_chunk(k, _):
        base = k * CHUNK
        pltpu.sync_copy(xh.at[c, pl.ds(nbase + base, CHUNK)], rowbuf0)

        def _row(i, _):
            d = _dis_splat(base + i)
            for v in range(VPR):
                sl = pl.ds(v * L, L)
                rowbuf0[i, sl] = d * rowbuf0[i, sl]
            return 0

        lax.fori_loop(0, CHUNK, _row, 0)
        pltpu.sync_copy(rowbuf0, g_hbm.at[pl.ds(gbase + base, CHUNK)])
        return 0

    with jax.named_scope("dis_g0"):
        lax.fori_loop(0, NCHUNK, _g0_chunk, 0)

    with scope("bar2"):
        plsc.subcore_barrier()

    gbufs = (rowbuf0, rowbuf1, zbuf64, rowbuf3)
    gsems = (gsem0, gsem1, gsem2, gsem3)
    ssems = (ssem0, ssem1, ssem2, ssem3)

    def _edge_round():
        pltpu.async_copy(g_hbm.at[src_idx.at[0]], gbufs[0], gsems[0])
        pltpu.async_copy(g_hbm.at[src_idx.at[1]], gbufs[1], gsems[1])

        def _quad(k, _):
            for b in range(4):
                j = 4 * k + b
                b2 = (b + 2) % 4

                @pl.when(j + 2 < ROWS)
                def _refill():
                    @pl.when(j >= 2)
                    def _wait_prev_scatter():
                        pltpu.make_async_copy(
                            gbufs[b2], acc_sp.at[dst_idx.at[0]],
                            ssems[b2]).wait()

                    pltpu.async_copy(g_hbm.at[src_idx.at[j + 2]], gbufs[b2],
                                     gsems[b2])

                pltpu.make_async_copy(g_hbm.at[src_idx.at[0]], gbufs[b],
                                      gsems[b]).wait()
                pltpu.async_copy(gbufs[b], acc_sp.at[dst_idx.at[j]], ssems[b],
                                 add=True)
            return 0

        lax.fori_loop(0, ROWS // 4, _quad, 0)
        for b in range(4):
            pltpu.make_async_copy(gbufs[b], acc_sp.at[dst_idx.at[0]],
                                  ssems[b]).wait()

    with jax.named_scope("hop1"):
        _edge_round()

    with scope("bar3"):
        plsc.subcore_barrier()

    lax.fori_loop(0, CHUNK, _init_row, 0)

    def _g1_chunk(k, _):
        base = k * CHUNK
        pltpu.sync_copy(acc_sp.at[pl.ds(nbase + base, CHUNK)], rowbuf1)
        pltpu.sync_copy(g_hbm.at[pl.ds(gbase + base, CHUNK)], rowbuf0)

        def _row(i, _):
            d = _dis_splat(base + i)
            d2 = d * d
            for v in range(VPR):
                sl = pl.ds(v * L, L)
                rowbuf0[i, sl] = d2 * (rowbuf1[i, sl] + rowbuf0[i, sl])
            return 0

        lax.fori_loop(0, CHUNK, _row, 0)
        pltpu.sync_copy(rowbuf0, g_hbm.at[pl.ds(gbase + base, CHUNK)])
        pltpu.sync_copy(zbuf64, acc_sp.at[pl.ds(nbase + base, CHUNK)])
        return 0

    with jax.named_scope("g1"):
        lax.fori_loop(0, NCHUNK, _g1_chunk, 0)

    with scope("bar4"):
        plsc.subcore_barrier()

    with jax.named_scope("hop2"):
        _edge_round()

    with scope("bar5"):
        plsc.subcore_barrier()

    def _out_chunk(k, _):
        base = k * CHUNK
        pltpu.sync_copy(acc_sp.at[pl.ds(nbase + base, CHUNK)], rowbuf1)
        pltpu.sync_copy(g_hbm.at[pl.ds(gbase + base, CHUNK)], rowbuf0)

        def _row(i, _):
            d = _dis_splat(base + i)
            for v in range(VPR):
                sl = pl.ds(v * L, L)
                rowbuf1[i, sl] = d * (rowbuf1[i, sl] + rowbuf0[i, sl])
            return 0

        lax.fori_loop(0, CHUNK, _row, 0)
        pltpu.sync_copy(rowbuf1, out.at[c, pl.ds(nbase + base, CHUNK)])
        return 0

    with jax.named_scope("outp"):
        lax.fori_loop(0, NCHUNK, _out_chunk, 0)


def _propagate_sc(xh, srcp, dstp):
    mesh = plsc.VectorSubcoreMesh(core_axis_name="c", subcore_axis_name="s")
    out, _ = pl.kernel(
        _sgc_body,
        out_type=(
            jax.ShapeDtypeStruct((NC, NPAD, DH), jnp.float32),
            jax.ShapeDtypeStruct((NC * NPAD, DH), jnp.float32),
        ),
        mesh=mesh,
        compiler_params=pltpu.CompilerParams(needs_layout_passes=False,
                                             use_tc_tiling_on_sc=False),
        scratch_types=[
            pltpu.VMEM_SHARED((NPAD, DH), jnp.float32),
            pltpu.VMEM_SHARED((NPAD, L), jnp.float32),
            pltpu.VMEM((ROWS, CHUNK), jnp.int32),
            pltpu.VMEM((ROWS, CHUNK), jnp.int32),
            pltpu.VMEM((CHUNK, L), jnp.float32),
            pltpu.VMEM((RPT,), jnp.float32),
            pltpu.VMEM((CHUNK, DH), jnp.float32),
            pltpu.VMEM((CHUNK, DH), jnp.float32),
            pltpu.VMEM((CHUNK, DH), jnp.float32),
            pltpu.VMEM((CHUNK, DH), jnp.float32),
            pltpu.VMEM((CHUNK // 2, L), jnp.float32),
            pltpu.VMEM((CHUNK, L), jnp.float32),
            pltpu.SemaphoreType.DMA,
            pltpu.SemaphoreType.DMA,
            pltpu.SemaphoreType.DMA,
            pltpu.SemaphoreType.DMA,
            pltpu.SemaphoreType.DMA,
            pltpu.SemaphoreType.DMA,
            pltpu.SemaphoreType.DMA,
            pltpu.SemaphoreType.DMA,
            pltpu.SemaphoreType.DMA,
        ],
    )(xh, srcp, dstp)
    return out


def _linsoftmax_body(h_ref, wt_ref, b_ref, o_ref):
    logits = jnp.dot(h_ref[...], wt_ref[...],
                     preferred_element_type=jnp.float32) + b_ref[...]
    m = jnp.max(logits, axis=1, keepdims=True)
    ex = jnp.exp(logits - m)
    o_ref[...] = ex / jnp.sum(ex, axis=1, keepdims=True)


def _linsoftmax_tc(h, wt, b2):
    blk = 1000
    grid = N // blk
    return pl.pallas_call(
        _linsoftmax_body,
        grid=(grid,),
        in_specs=[
            pl.BlockSpec((blk, D), lambda i: (i, 0)),
            pl.BlockSpec((D, C), lambda i: (0, 0)),
            pl.BlockSpec((1, C), lambda i: (0, 0)),
        ],
        out_specs=pl.BlockSpec((blk, C), lambda i: (i, 0)),
        out_shape=jax.ShapeDtypeStruct((N, C), jnp.float32),
    )(h, wt, b2)


def kernel(x, edge_index, W, b):
    src = edge_index[0]
    dst = edge_index[1]
    pad = N + (jnp.arange(ETOT - E, dtype=jnp.int32) % (NPAD - N))
    src1 = jnp.concatenate([src, pad])
    srcp = jnp.stack([src1, src1 + NPAD]).reshape(NC, NS * ROWS, CHUNK)
    dstp = jnp.concatenate([dst, pad]).reshape(NS * ROWS, CHUNK)
    xp = jnp.pad(x, ((0, NPAD - N), (0, 0)))
    xh = xp.reshape(NPAD, NC, DH).transpose(1, 0, 2)

    halves = _propagate_sc(xh, srcp, dstp)
    h2 = halves[:, :N].transpose(1, 0, 2).reshape(N, D)

    return _linsoftmax_tc(h2, W.T, b.reshape(1, C))


# speedup vs baseline: 2.2273x; 1.1006x over previous
import jax
import jax.numpy as jnp
from jax import lax
from jax.experimental import pallas as pl
from jax.experimental.pallas import tpu as pltpu
from jax.experimental.pallas import tpu_sc as plsc

N = 10000
D = 128
E = 320000
C = 64

NC = 2
NS = 16
L = 16
DH = D // NC

CHUNK = 128
ROWS = 160
EPT = ROWS * CHUNK
ETOT = NS * EPT

RPT = 640
NPAD = NS * RPT
NCHUNK = RPT // CHUNK
VPR = DH // L


def _rsqrt_newton(x):
    i = plsc.bitcast(x, jnp.int32)
    i = jnp.int32(0x5F3759DF) - (i >> 1)
    y = plsc.bitcast(i, jnp.float32)
    for _ in range(3):
        y = y * (jnp.float32(1.5) - jnp.float32(0.5) * x * y * y)
    return y


def _sgc_body(xh, edges, out, g_hbm,
              acc_sp, deg_sp,
              src_idx, dst_idx, degloc, disloc,
              rowbuf0, rowbuf1, rowbuf3, zbuf64, zbuf16, onesbuf,
              gsem0, gsem1, gsem2, gsem3, ssem0, ssem1, ssem2, ssem3, dsem):
    c = lax.axis_index("c")
    s = lax.axis_index("s")
    nbase = s * RPT
    gbase = c * NPAD + nbase
    scope = jax.named_scope

    zero16 = jnp.zeros((L,), jnp.float32)
    e0 = jnp.where(lax.iota(jnp.int32, L) == 0, jnp.float32(1.0),
                   jnp.float32(0.0))

    _sc_init = scope("init")
    _sc_init.__enter__()

    def _init_row(i, _):
        onesbuf[i, :] = e0
        for v in range(VPR):
            zbuf64[i, pl.ds(v * L, L)] = zero16
        return 0

    lax.fori_loop(0, CHUNK, _init_row, 0)

    def _init_z16(i, _):
        zbuf16[i, :] = zero16
        return 0

    lax.fori_loop(0, CHUNK // 2, _init_z16, 0)

    def _zero_chunk(k, _):
        pltpu.sync_copy(zbuf16,
                        deg_sp.at[pl.ds(nbase + k * (CHUNK // 2), CHUNK // 2)])
        return 0

    lax.fori_loop(0, 2 * NCHUNK, _zero_chunk, 0)

    def _zero_acc_chunk(k, _):
        pltpu.sync_copy(zbuf64, acc_sp.at[pl.ds(nbase + k * CHUNK, CHUNK)])
        return 0

    lax.fori_loop(0, NCHUNK, _zero_acc_chunk, 0)

    pltpu.sync_copy(edges.at[0, pl.ds(s * ROWS, ROWS)], src_idx)
    pltpu.sync_copy(edges.at[1, pl.ds(s * ROWS, ROWS)], dst_idx)

    coff = jnp.broadcast_to((c * NPAD).astype(jnp.int32), (L,))

    def _off_row(r, _):
        for v in range(CHUNK // L):
            sl = pl.ds(v * L, L)
            src_idx[r, sl] = src_idx[r, sl] + coff
        return 0

    lax.fori_loop(0, ROWS, _off_row, 0)
    _sc_init.__exit__(None, None, None)

    with scope("bar0"):
        plsc.subcore_barrier()

    def _deg_fire(j, _):
        pltpu.async_copy(onesbuf, deg_sp.at[dst_idx.at[j]], dsem, add=True)
        return 0

    def _deg_drain(j, _):
        pltpu.make_async_copy(onesbuf, deg_sp.at[dst_idx.at[j]], dsem).wait()
        return 0

    with jax.named_scope("deg"):
        lax.fori_loop(0, ROWS, _deg_fire, 0)
        lax.fori_loop(0, ROWS, _deg_drain, 0)

    with scope("bar1"):
        plsc.subcore_barrier()

    def _dis_chunk(k, _):
        pltpu.sync_copy(deg_sp.at[pl.ds(nbase + k * CHUNK, CHUNK)], degloc)

        def _grp(g, _):
            ridx = g * L + lax.iota(jnp.int32, L)
            cidx = jnp.zeros((L,), jnp.int32)
            cnt = plsc.load_gather(degloc, [ridx, cidx])
            disloc[pl.ds(k * CHUNK + g * L, L)] = _rsqrt_newton(
                cnt + jnp.float32(1.0))
            return 0

        lax.fori_loop(0, CHUNK // L, _grp, 0)
        return 0

    lax.fori_loop(0, NCHUNK, _dis_chunk, 0)

    def _dis_splat(r):
        return plsc.load_gather(disloc, [jnp.full((L,), r, jnp.int32)])

    def _g0_chunk(k, _):
        base = k * CHUNK
        pltpu.sync_copy(xh.at[c, pl.ds(nbase + base, CHUNK)], rowbuf0)

        def _row(i, _):
            d = _dis_splat(base + i)
            for v in range(VPR):
                sl = pl.ds(v * L, L)
                rowbuf0[i, sl] = d * rowbuf0[i, sl]
            return 0

        lax.fori_loop(0, CHUNK, _row, 0)
        pltpu.sync_copy(rowbuf0, g_hbm.at[pl.ds(gbase + base, CHUNK)])
        return 0

    with jax.named_scope("dis_g0"):
        lax.fori_loop(0, NCHUNK, _g0_chunk, 0)

    with scope("bar2"):
        plsc.subcore_barrier()

    gbufs = (rowbuf0, rowbuf1, zbuf64, rowbuf3)
    gsems = (gsem0, gsem1, gsem2, gsem3)
    ssems = (ssem0, ssem1, ssem2, ssem3)

    def _edge_round():
        pltpu.async_copy(g_hbm.at[src_idx.at[0]], gbufs[0], gsems[0])
        pltpu.async_copy(g_hbm.at[src_idx.at[1]], gbufs[1], gsems[1])

        def _quad(k, _):
            for b in range(4):
                j = 4 * k + b
                b2 = (b + 2) % 4

                @pl.when(j + 2 < ROWS)
                def _refill():
                    @pl.when(j >= 2)
                    def _wait_prev_scatter():
                        pltpu.make_async_copy(
                            gbufs[b2], acc_sp.at[dst_idx.at[0]],
                            ssems[b2]).wait()

                    pltpu.async_copy(g_hbm.at[src_idx.at[j + 2]], gbufs[b2],
                                     gsems[b2])

                pltpu.make_async_copy(g_hbm.at[src_idx.at[0]], gbufs[b],
                                      gsems[b]).wait()
                pltpu.async_copy(gbufs[b], acc_sp.at[dst_idx.at[j]], ssems[b],
                                 add=True)
            return 0

        lax.fori_loop(0, ROWS // 4, _quad, 0)
        for b in range(4):
            pltpu.make_async_copy(gbufs[b], acc_sp.at[dst_idx.at[0]],
                                  ssems[b]).wait()

    with jax.named_scope("hop1"):
        _edge_round()

    with scope("bar3"):
        plsc.subcore_barrier()

    lax.fori_loop(0, CHUNK, _init_row, 0)

    def _g1_chunk(k, _):
        base = k * CHUNK
        pltpu.sync_copy(acc_sp.at[pl.ds(nbase + base, CHUNK)], rowbuf1)
        pltpu.sync_copy(g_hbm.at[pl.ds(gbase + base, CHUNK)], rowbuf0)

        def _row(i, _):
            d = _dis_splat(base + i)
            d2 = d * d
            for v in range(VPR):
                sl = pl.ds(v * L, L)
                rowbuf0[i, sl] = d2 * (rowbuf1[i, sl] + rowbuf0[i, sl])
            return 0

        lax.fori_loop(0, CHUNK, _row, 0)
        pltpu.sync_copy(rowbuf0, g_hbm.at[pl.ds(gbase + base, CHUNK)])
        pltpu.sync_copy(zbuf64, acc_sp.at[pl.ds(nbase + base, CHUNK)])
        return 0

    with jax.named_scope("g1"):
        lax.fori_loop(0, NCHUNK, _g1_chunk, 0)

    with scope("bar4"):
        plsc.subcore_barrier()

    with jax.named_scope("hop2"):
        _edge_round()

    with scope("bar5"):
        plsc.subcore_barrier()

    def _out_chunk(k, _):
        base = k * CHUNK
        pltpu.sync_copy(acc_sp.at[pl.ds(nbase + base, CHUNK)], rowbuf1)
        pltpu.sync_copy(g_hbm.at[pl.ds(gbase + base, CHUNK)], rowbuf0)

        def _row(i, _):
            d = _dis_splat(base + i)
            for v in range(VPR):
                sl = pl.ds(v * L, L)
                rowbuf1[i, sl] = d * (rowbuf1[i, sl] + rowbuf0[i, sl])
            return 0

        lax.fori_loop(0, CHUNK, _row, 0)
        pltpu.sync_copy(rowbuf1, out.at[c, pl.ds(nbase + base, CHUNK)])
        return 0

    with jax.named_scope("outp"):
        lax.fori_loop(0, NCHUNK, _out_chunk, 0)


def _propagate_sc(xh, edges):
    mesh = plsc.VectorSubcoreMesh(core_axis_name="c", subcore_axis_name="s")
    out, _ = pl.kernel(
        _sgc_body,
        out_type=(
            jax.ShapeDtypeStruct((NC, NPAD, DH), jnp.float32),
            jax.ShapeDtypeStruct((NC * NPAD, DH), jnp.float32),
        ),
        mesh=mesh,
        compiler_params=pltpu.CompilerParams(needs_layout_passes=False,
                                             use_tc_tiling_on_sc=False),
        scratch_types=[
            pltpu.VMEM_SHARED((NPAD, DH), jnp.float32),
            pltpu.VMEM_SHARED((NPAD, L), jnp.float32),
            pltpu.VMEM((ROWS, CHUNK), jnp.int32),
            pltpu.VMEM((ROWS, CHUNK), jnp.int32),
            pltpu.VMEM((CHUNK, L), jnp.float32),
            pltpu.VMEM((RPT,), jnp.float32),
            pltpu.VMEM((CHUNK, DH), jnp.float32),
            pltpu.VMEM((CHUNK, DH), jnp.float32),
            pltpu.VMEM((CHUNK, DH), jnp.float32),
            pltpu.VMEM((CHUNK, DH), jnp.float32),
            pltpu.VMEM((CHUNK // 2, L), jnp.float32),
            pltpu.VMEM((CHUNK, L), jnp.float32),
            pltpu.SemaphoreType.DMA,
            pltpu.SemaphoreType.DMA,
            pltpu.SemaphoreType.DMA,
            pltpu.SemaphoreType.DMA,
            pltpu.SemaphoreType.DMA,
            pltpu.SemaphoreType.DMA,
            pltpu.SemaphoreType.DMA,
            pltpu.SemaphoreType.DMA,
            pltpu.SemaphoreType.DMA,
        ],
    )(xh, edges)
    return out


def _linsoftmax_body(ha_ref, hb_ref, wt_ref, b_ref, o_ref):
    logits = (jnp.dot(ha_ref[0], wt_ref[0:DH, :],
                      preferred_element_type=jnp.float32)
              + jnp.dot(hb_ref[0], wt_ref[DH:D, :],
                        preferred_element_type=jnp.float32)
              + b_ref[...])
    m = jnp.max(logits, axis=1, keepdims=True)
    ex = jnp.exp(logits - m)
    o_ref[...] = ex / jnp.sum(ex, axis=1, keepdims=True)


def _linsoftmax_tc(halves, wt, b2):
    blk = 1000
    grid = N // blk
    return pl.pallas_call(
        _linsoftmax_body,
        grid=(grid,),
        in_specs=[
            pl.BlockSpec((1, blk, DH), lambda i: (0, i, 0)),
            pl.BlockSpec((1, blk, DH), lambda i: (1, i, 0)),
            pl.BlockSpec((D, C), lambda i: (0, 0)),
            pl.BlockSpec((1, C), lambda i: (0, 0)),
        ],
        out_specs=pl.BlockSpec((blk, C), lambda i: (i, 0)),
        out_shape=jax.ShapeDtypeStruct((N, C), jnp.float32),
    )(halves, halves, wt, b2)


def kernel(x, edge_index, W, b):
    src = edge_index[0]
    dst = edge_index[1]
    pad = N + (jnp.arange(ETOT - E, dtype=jnp.int32) % (NPAD - N))
    edges = jnp.concatenate(
        [edge_index, jnp.stack([pad, pad])], axis=1,
    ).reshape(2, NS * ROWS, CHUNK)
    xp = jnp.pad(x, ((0, NPAD - N), (0, 0)))
    xh = xp.reshape(NPAD, NC, DH).transpose(1, 0, 2)

    halves = _propagate_sc(xh, edges)
    return _linsoftmax_tc(halves, W.T, b.reshape(1, C))


# speedup vs baseline: 2.3638x; 1.0613x over previous
import jax
import jax.numpy as jnp
from jax import lax
from jax.experimental import pallas as pl
from jax.experimental.pallas import tpu as pltpu
from jax.experimental.pallas import tpu_sc as plsc

N = 10000
D = 128
E = 320000
C = 64

NC = 2
NS = 16
L = 16
DH = D // NC

CHUNK = 128
ROWS = 160
EPT = ROWS * CHUNK
ETOT = NS * EPT

RPT = 640
NPAD = NS * RPT
NCHUNK = RPT // CHUNK
VPR = DH // L


def _rsqrt_newton(x):
    i = plsc.bitcast(x, jnp.int32)
    i = jnp.int32(0x5F3759DF) - (i >> 1)
    y = plsc.bitcast(i, jnp.float32)
    for _ in range(3):
        y = y * (jnp.float32(1.5) - jnp.float32(0.5) * x * y * y)
    return y


def _sgc_body(xh, edges, out, g_hbm,
              acc_sp, deg_sp,
              src_idx, dst_idx, degloc, disloc,
              rowbuf0, rowbuf1, rowbuf3, zbuf64, zbuf16, onesbuf,
              gsem0, gsem1, gsem2, gsem3, ssem0, ssem1, ssem2, ssem3, dsem):
    c = lax.axis_index("c")
    s = lax.axis_index("s")
    nbase = s * RPT
    gbase = c * NPAD + nbase
    scope = jax.named_scope

    zero16 = jnp.zeros((L,), jnp.float32)
    e0 = jnp.where(lax.iota(jnp.int32, L) == 0, jnp.float32(1.0),
                   jnp.float32(0.0))

    _sc_init = scope("init")
    _sc_init.__enter__()

    def _init_row(i, _):
        onesbuf[i, :] = e0
        for v in range(VPR):
            zbuf64[i, pl.ds(v * L, L)] = zero16
        return 0

    lax.fori_loop(0, CHUNK, _init_row, 0)

    def _init_z16(i, _):
        zbuf16[i, :] = zero16
        return 0

    lax.fori_loop(0, CHUNK // 2, _init_z16, 0)

    def _zero_chunk(k, _):
        pltpu.sync_copy(zbuf16,
                        deg_sp.at[pl.ds(nbase + k * (CHUNK // 2), CHUNK // 2)])
        return 0

    lax.fori_loop(0, 2 * NCHUNK, _zero_chunk, 0)

    def _zero_acc_chunk(k, _):
        pltpu.sync_copy(zbuf64, acc_sp.at[pl.ds(nbase + k * CHUNK, CHUNK)])
        return 0

    lax.fori_loop(0, NCHUNK, _zero_acc_chunk, 0)

    pltpu.sync_copy(edges.at[0, pl.ds(s * ROWS, ROWS)], src_idx)
    pltpu.sync_copy(edges.at[1, pl.ds(s * ROWS, ROWS)], dst_idx)

    coff = jnp.broadcast_to((c * NPAD).astype(jnp.int32), (L,))

    def _off_row(r, _):
        for v in range(CHUNK // L):
            sl = pl.ds(v * L, L)
            src_idx[r, sl] = src_idx[r, sl] + coff
        return 0

    lax.fori_loop(0, ROWS, _off_row, 0)
    _sc_init.__exit__(None, None, None)

    with scope("bar0"):
        plsc.subcore_barrier()

    def _deg_fire(j, _):
        pltpu.async_copy(onesbuf, deg_sp.at[dst_idx.at[j]], dsem, add=True)
        return 0

    def _deg_drain(j, _):
        pltpu.make_async_copy(onesbuf, deg_sp.at[dst_idx.at[j]], dsem).wait()
        return 0

    with jax.named_scope("deg"):
        lax.fori_loop(0, ROWS, _deg_fire, 0)
        lax.fori_loop(0, ROWS, _deg_drain, 0)

    with scope("bar1"):
        plsc.subcore_barrier()

    def _dis_chunk(k, _):
        pltpu.sync_copy(deg_sp.at[pl.ds(nbase + k * CHUNK, CHUNK)], degloc)

        def _grp(g, _):
            ridx = g * L + lax.iota(jnp.int32, L)
            cidx = jnp.zeros((L,), jnp.int32)
            cnt = plsc.load_gather(degloc, [ridx, cidx])
            disloc[pl.ds(k * CHUNK + g * L, L)] = _rsqrt_newton(
                cnt + jnp.float32(1.0))
            return 0

        lax.fori_loop(0, CHUNK // L, _grp, 0)
        return 0

    lax.fori_loop(0, NCHUNK, _dis_chunk, 0)

    def _dis_splat(r):
        return plsc.load_gather(disloc, [jnp.full((L,), r, jnp.int32)])

    def _g0_chunk(k, _):
        base = k * CHUNK
        pltpu.sync_copy(
            xh.at[pl.ds(nbase + base, CHUNK), pl.ds(c * DH, DH)], rowbuf0)

        def _row(i, _):
            d = _dis_splat(base + i)
            for v in range(VPR):
                sl = pl.ds(v * L, L)
                rowbuf0[i, sl] = d * rowbuf0[i, sl]
            return 0

        lax.fori_loop(0, CHUNK, _row, 0)
        pltpu.sync_copy(rowbuf0, g_hbm.at[pl.ds(gbase + base, CHUNK)])
        return 0

    with jax.named_scope("dis_g0"):
        lax.fori_loop(0, NCHUNK, _g0_chunk, 0)

    with scope("bar2"):
        plsc.subcore_barrier()

    gbufs = (rowbuf0, rowbuf1, zbuf64, rowbuf3)
    gsems = (gsem0, gsem1, gsem2, gsem3)
    ssems = (ssem0, ssem1, ssem2, ssem3)

    def _edge_round():
        pltpu.async_copy(g_hbm.at[src_idx.at[0]], gbufs[0], gsems[0])
        pltpu.async_copy(g_hbm.at[src_idx.at[1]], gbufs[1], gsems[1])

        def _quad(k, _):
            for b in range(4):
                j = 4 * k + b
                b2 = (b + 2) % 4

                @pl.when(j + 2 < ROWS)
                def _refill():
                    @pl.when(j >= 2)
                    def _wait_prev_scatter():
                        pltpu.make_async_copy(
                            gbufs[b2], acc_sp.at[dst_idx.at[0]],
                            ssems[b2]).wait()

                    pltpu.async_copy(g_hbm.at[src_idx.at[j + 2]], gbufs[b2],
                                     gsems[b2])

                pltpu.make_async_copy(g_hbm.at[src_idx.at[0]], gbufs[b],
                                      gsems[b]).wait()
                pltpu.async_copy(gbufs[b], acc_sp.at[dst_idx.at[j]], ssems[b],
                                 add=True)
            return 0

        lax.fori_loop(0, ROWS // 4, _quad, 0)
        for b in range(4):
            pltpu.make_async_copy(gbufs[b], acc_sp.at[dst_idx.at[0]],
                                  ssems[b]).wait()

    with jax.named_scope("hop1"):
        _edge_round()

    with scope("bar3"):
        plsc.subcore_barrier()

    lax.fori_loop(0, CHUNK, _init_row, 0)

    def _g1_chunk(k, _):
        base = k * CHUNK
        pltpu.sync_copy(acc_sp.at[pl.ds(nbase + base, CHUNK)], rowbuf1)
        pltpu.sync_copy(g_hbm.at[pl.ds(gbase + base, CHUNK)], rowbuf0)

        def _row(i, _):
            d = _dis_splat(base + i)
            d2 = d * d
            for v in range(VPR):
                sl = pl.ds(v * L, L)
                rowbuf0[i, sl] = d2 * (rowbuf1[i, sl] + rowbuf0[i, sl])
            return 0

        lax.fori_loop(0, CHUNK, _row, 0)
        pltpu.sync_copy(rowbuf0, g_hbm.at[pl.ds(gbase + base, CHUNK)])
        pltpu.sync_copy(zbuf64, acc_sp.at[pl.ds(nbase + base, CHUNK)])
        return 0

    with jax.named_scope("g1"):
        lax.fori_loop(0, NCHUNK, _g1_chunk, 0)

    with scope("bar4"):
        plsc.subcore_barrier()

    with jax.named_scope("hop2"):
        _edge_round()

    with scope("bar5"):
        plsc.subcore_barrier()

    def _out_chunk(k, _):
        base = k * CHUNK
        pltpu.sync_copy(acc_sp.at[pl.ds(nbase + base, CHUNK)], rowbuf1)
        pltpu.sync_copy(g_hbm.at[pl.ds(gbase + base, CHUNK)], rowbuf0)

        def _row(i, _):
            d = _dis_splat(base + i)
            for v in range(VPR):
                sl = pl.ds(v * L, L)
                rowbuf1[i, sl] = d * (rowbuf1[i, sl] + rowbuf0[i, sl])
            return 0

        lax.fori_loop(0, CHUNK, _row, 0)
        pltpu.sync_copy(rowbuf1, out.at[c, pl.ds(nbase + base, CHUNK)])
        return 0

    with jax.named_scope("outp"):
        lax.fori_loop(0, NCHUNK, _out_chunk, 0)


def _propagate_sc(xh, edges):
    mesh = plsc.VectorSubcoreMesh(core_axis_name="c", subcore_axis_name="s")
    out, _ = pl.kernel(
        _sgc_body,
        out_type=(
            jax.ShapeDtypeStruct((NC, NPAD, DH), jnp.float32),
            jax.ShapeDtypeStruct((NC * NPAD, DH), jnp.float32),
        ),
        mesh=mesh,
        compiler_params=pltpu.CompilerParams(needs_layout_passes=False,
                                             use_tc_tiling_on_sc=False),
        scratch_types=[
            pltpu.VMEM_SHARED((NPAD, DH), jnp.float32),
            pltpu.VMEM_SHARED((NPAD, L), jnp.float32),
            pltpu.VMEM((ROWS, CHUNK), jnp.int32),
            pltpu.VMEM((ROWS, CHUNK), jnp.int32),
            pltpu.VMEM((CHUNK, L), jnp.float32),
            pltpu.VMEM((RPT,), jnp.float32),
            pltpu.VMEM((CHUNK, DH), jnp.float32),
            pltpu.VMEM((CHUNK, DH), jnp.float32),
            pltpu.VMEM((CHUNK, DH), jnp.float32),
            pltpu.VMEM((CHUNK, DH), jnp.float32),
            pltpu.VMEM((CHUNK // 2, L), jnp.float32),
            pltpu.VMEM((CHUNK, L), jnp.float32),
            pltpu.SemaphoreType.DMA,
            pltpu.SemaphoreType.DMA,
            pltpu.SemaphoreType.DMA,
            pltpu.SemaphoreType.DMA,
            pltpu.SemaphoreType.DMA,
            pltpu.SemaphoreType.DMA,
            pltpu.SemaphoreType.DMA,
            pltpu.SemaphoreType.DMA,
            pltpu.SemaphoreType.DMA,
        ],
    )(xh, edges)
    return out


def _linsoftmax_body(ha_ref, hb_ref, wt_ref, b_ref, o_ref):
    logits = (jnp.dot(ha_ref[0], wt_ref[0:DH, :],
                      preferred_element_type=jnp.float32)
              + jnp.dot(hb_ref[0], wt_ref[DH:D, :],
                        preferred_element_type=jnp.float32)
              + b_ref[...])
    m = jnp.max(logits, axis=1, keepdims=True)
    ex = jnp.exp(logits - m)
    o_ref[...] = ex / jnp.sum(ex, axis=1, keepdims=True)


def _linsoftmax_tc(halves, wt, b2):
    blk = 1000
    grid = N // blk
    return pl.pallas_call(
        _linsoftmax_body,
        grid=(grid,),
        in_specs=[
            pl.BlockSpec((1, blk, DH), lambda i: (0, i, 0)),
            pl.BlockSpec((1, blk, DH), lambda i: (1, i, 0)),
            pl.BlockSpec((D, C), lambda i: (0, 0)),
            pl.BlockSpec((1, C), lambda i: (0, 0)),
        ],
        out_specs=pl.BlockSpec((blk, C), lambda i: (i, 0)),
        out_shape=jax.ShapeDtypeStruct((N, C), jnp.float32),
    )(halves, halves, wt, b2)


def kernel(x, edge_index, W, b):
    src = edge_index[0]
    dst = edge_index[1]
    pad = N + (jnp.arange(ETOT - E, dtype=jnp.int32) % (NPAD - N))
    edges = jnp.concatenate(
        [edge_index, jnp.stack([pad, pad])], axis=1,
    ).reshape(2, NS * ROWS, CHUNK)
    xp = jnp.pad(x, ((0, NPAD - N), (0, 0)))

    halves = _propagate_sc(xp, edges)
    return _linsoftmax_tc(halves, W.T, b.reshape(1, C))
